# baseline (speedup 1.0000x reference)
import functools

import jax
import jax.numpy as jnp
from jax import lax
from jax.experimental import pallas as pl
from jax.experimental.pallas import tpu as pltpu
from jax.experimental.pallas import tpu_sc as plsc

N = 10000
NP = 10240
E = 160000
EP = 163840
DIN = 162
DH = 512
DHALF = 96
DPK = DHALF // 2
ONES_COL = 66
CH = 128
NCHUNK = EP // CH
NS = 16
CPT = NCHUNK // NS
RPT = NP // NS
NSLOT = 2
NTURN = CPT // NSLOT
BLK = 1000
ABLK = 1024


def _rne_hi16(x):
    b = lax.bitcast_convert_type(x, jnp.int32)
    r = b + 0x7FFF + lax.bitwise_and(lax.shift_right_logical(b, 16), 1)
    return lax.bitwise_and(r, jnp.int32(-65536))


def _seg_body(tb0, tb1, tf0, tf1, uu, vv, z0, z1, w0, w1, zp0, zp1,
              ub, vb, rbf, rows, gsem, ssem, acc):
    c = lax.axis_index("c")
    s = lax.axis_index("s")

    def bslot(b):
        return rbf.at[pl.ds(b * CH, CH)]

    def rslot(b):
        return rows.at[pl.ds(b * CH, CH)]

    def run(tab, tabf, zout, wout, zpk):
        c0 = pl.multiple_of(s * CPT, 8)
        pltpu.sync_copy(uu.at[pl.ds(c0, CPT)], ub)
        pltpu.sync_copy(vv.at[pl.ds(c0, CPT)], vb)
        def initb(kb, carry):
            r2 = pl.multiple_of(s * RPT + kb * CH, 8)
            pltpu.sync_copy(tabf.at[pl.ds(r2, CH)], rslot(0))
            pltpu.sync_copy(rslot(0), acc.at[pl.ds(r2, CH)])
            return carry

        lax.fori_loop(0, RPT // CH, initb, 0)
        plsc.subcore_barrier()

        def widen(b):
            base = b * CH

            def row(r, carry):
                for g in range(DPK // 16):
                    xi = rbf[base + r, pl.ds(16 * g, 16)]
                    lo = lax.bitcast_convert_type(
                        lax.shift_left(xi, 16), jnp.float32)
                    hi = lax.bitcast_convert_type(
                        lax.bitwise_and(xi, jnp.int32(-65536)), jnp.float32)
                    rows[base + r, pl.ds(16 * g, 16)] = lo
                    rows[base + r, pl.ds(DPK + 16 * g, 16)] = hi
                return carry

            lax.fori_loop(0, CH, row, 0)

        def pass_loop(gtab):
            def g_start(k, b):
                pltpu.async_copy(gtab.at[ub.at[k]], bslot(b), gsem.at[b])

            def g_wait(k, b):
                pltpu.make_async_copy(gtab.at[ub.at[k]], bslot(b),
                                      gsem.at[b]).wait()

            def s_start(k, b):
                pltpu.async_copy(rslot(b), acc.at[vb.at[k]], ssem.at[b],
                                 add=True)

            def s_wait(k, b):
                pltpu.make_async_copy(rslot(b), acc.at[vb.at[k]],
                                      ssem.at[b]).wait()

            g_start(0, 0)

            def turn(j, carry):
                for b in range(NSLOT):
                    k = j * NSLOT + b
                    g_wait(k, b)

                    @pl.when(k >= 1)
                    def _():
                        s_wait(k - 1, 1 - b)

                    @pl.when(k + 1 < CPT)
                    def _():
                        g_start(k + 1, 1 - b)

                    widen(b)
                    s_start(k, b)
                return carry

            lax.fori_loop(0, NTURN, turn, 0)
            s_wait(CPT - 1, (CPT - 1) % NSLOT)

        pass_loop(tab)
        plsc.subcore_barrier()

        def zblock(kb, carry):
            r2 = pl.multiple_of(s * RPT + kb * CH, 8)
            pltpu.sync_copy(acc.at[pl.ds(r2, CH)], rslot(0))
            pltpu.sync_copy(rslot(0), zout.at[pl.ds(r2, CH)])

            def row(r, carry2):
                for g in range(DPK // 16):
                    lo = _rne_hi16(rows[r, pl.ds(16 * g, 16)])
                    hi = _rne_hi16(rows[r, pl.ds(DPK + 16 * g, 16)])
                    rbf[r, pl.ds(16 * g, 16)] = lax.bitwise_or(
                        lax.shift_right_logical(lo, 16), hi)
                return carry2

            lax.fori_loop(0, CH, row, 0)
            pltpu.sync_copy(rbf.at[pl.ds(0, CH)], zpk.at[pl.ds(r2, CH)])
            return carry

        lax.fori_loop(0, RPT // CH, zblock, 0)
        plsc.subcore_barrier()

        pass_loop(zpk)
        plsc.subcore_barrier()

        def woutb(kb, carry):
            r2 = pl.multiple_of(s * RPT + kb * CH, 8)
            pltpu.sync_copy(acc.at[pl.ds(r2, CH)], rslot(0))
            pltpu.sync_copy(rslot(0), wout.at[pl.ds(r2, CH)])
            return carry

        lax.fori_loop(0, RPT // CH, woutb, 0)

    @pl.when(c == 0)
    def _():
        run(tb0, tf0, z0, w0, zp0)

    @pl.when(c == 1)
    def _():
        run(tb1, tf1, z1, w1, zp1)


_seg = functools.partial(
    pl.kernel,
    out_type=(jax.ShapeDtypeStruct((NP, DHALF), jnp.float32),
              jax.ShapeDtypeStruct((NP, DHALF), jnp.float32),
              jax.ShapeDtypeStruct((NP, DHALF), jnp.float32),
              jax.ShapeDtypeStruct((NP, DHALF), jnp.float32),
              jax.ShapeDtypeStruct((NP, DPK), jnp.int32),
              jax.ShapeDtypeStruct((NP, DPK), jnp.int32)),
    mesh=plsc.VectorSubcoreMesh(core_axis_name="c", subcore_axis_name="s"),
    scratch_types=[
        pltpu.VMEM((CPT, CH), jnp.int32),
        pltpu.VMEM((CPT, CH), jnp.int32),
        pltpu.VMEM((NSLOT * CH, DPK), jnp.int32),
        pltpu.VMEM((NSLOT * CH, DHALF), jnp.float32),
        pltpu.SemaphoreType.DMA((NSLOT,)),
        pltpu.SemaphoreType.DMA((NSLOT,)),
        pltpu.VMEM_SHARED((NP, DHALF), jnp.float32),
    ],
    compiler_params=pltpu.CompilerParams(use_tc_tiling_on_sc=False),
)(_seg_body)


def _mm_body(za1, wa0, wa1, w1a, w1b, b1, w2, b2, out):
    t = jnp.dot(wa0[...], w1a[...], preferred_element_type=jnp.float32)
    t += jnp.dot(wa1[...], w1b[...], preferred_element_type=jnp.float32)
    t += za1[:, ONES_COL:ONES_COL + 1] * b1[...]
    out[...] = jnp.dot(t, w2[...], preferred_element_type=jnp.float32) + b2[...]


_mm = pl.pallas_call(
    _mm_body,
    grid=(N // BLK,),
    in_specs=[
        pl.BlockSpec((BLK, DHALF), lambda i: (i, 0)),
        pl.BlockSpec((BLK, DHALF), lambda i: (i, 0)),
        pl.BlockSpec((BLK, DHALF), lambda i: (i, 0)),
        pl.BlockSpec((DHALF, DH), lambda i: (0, 0)),
        pl.BlockSpec((DHALF, DH), lambda i: (0, 0)),
        pl.BlockSpec((1, DH), lambda i: (0, 0)),
        pl.BlockSpec((DH, DH), lambda i: (0, 0)),
        pl.BlockSpec((1, DH), lambda i: (0, 0)),
    ],
    out_specs=pl.BlockSpec((BLK, DH), lambda i: (i, 0)),
    out_shape=jax.ShapeDtypeStruct((N, DH), jnp.float32),
)


def _pack_cols(t):
    lo = lax.shift_right_logical(_rne_hi16(t[:, :DPK]), 16)
    hi = _rne_hi16(t[:, DPK:])
    return lax.bitwise_or(lo, hi)


def kernel(x, edge_index, W1, b1, W2, b2):
    pad = jnp.full((EP - E,), N, jnp.int32)
    u = jnp.concatenate([edge_index[0].astype(jnp.int32), pad]).reshape(
        NCHUNK, CH)
    v = jnp.concatenate([edge_index[1].astype(jnp.int32), pad]).reshape(
        NCHUNK, CH)
    xa0 = jnp.pad(x[:, :DHALF], ((0, NP - N), (0, 0)))
    xa1 = jnp.pad(
        jnp.concatenate([x[:, DHALF:], jnp.ones((N, 1), jnp.float32)],
                        axis=1),
        ((0, NP - N), (0, DHALF - (DIN - DHALF) - 1)))
    xb0 = _pack_cols(xa0)
    xb1 = _pack_cols(xa1)
    za0, za1, wa0, wa1, _, _ = _seg(xb0, xb1, xa0, xa1, u, v)
    w1a = W1[:DHALF]
    w1b = jnp.pad(W1[DHALF:], ((0, DHALF - (DIN - DHALF)), (0, 0)))
    return _mm(za1[:N], wa0[:N], wa1[:N], w1a, w1b, b1.reshape(1, DH), W2,
               b2.reshape(1, DH))
